# baseline (speedup 1.0000x reference)
import functools

import jax
import jax.numpy as jnp
from jax import lax
from jax.experimental import pallas as pl
from jax.experimental.pallas import tpu as pltpu
from jax.experimental.pallas import tpu_sc as plsc

N_QUANT = 256
B = 16
T = 8192
NB = B * N_QUANT * T
QT = N_QUANT * T
CHUNK = NB // 32
ZW = 65536
NZ = CHUNK // ZW
TH = T // 2
NSCAT = TH // 128


def _sc_body(idx_hbm, out_hbm, zbuf, idx_v, offs_v, vals_v, zsem, ssem):
    c = lax.axis_index("c")
    s = lax.axis_index("s")
    b = c * 8 + s // 2
    half = s % 2

    zeros = jnp.zeros((16,), jnp.float32)
    ones = jnp.ones((16,), jnp.float32)

    def fill_zeros(i, carry):
        zbuf[pl.ds(i * 16, 16)] = zeros
        return carry

    lax.fori_loop(0, ZW // 16, fill_zeros, 0)

    pltpu.sync_copy(idx_hbm.at[b, pl.ds(half * TH, TH)], idx_v)
    base = b * QT + half * TH
    iota = lax.iota(jnp.int32, 16)
    for j in range(NSCAT):
        def fill_row(k, carry, j=j):
            v = idx_v[pl.ds(j * 128 + k * 16, 16)]
            offs_v[j, pl.ds(k * 16, 16)] = v * T + (base + j * 128 + k * 16) + iota
            vals_v[j, pl.ds(k * 16, 16)] = ones
            return carry

        lax.fori_loop(0, 8, fill_row, 0)

    zbase = b * QT + half * CHUNK
    zcopies = [
        pltpu.async_copy(zbuf, out_hbm.at[pl.ds(zbase + j * ZW, ZW)], zsem)
        for j in range(NZ)
    ]
    for cp in zcopies:
        cp.wait()

    plsc.subcore_barrier()

    scopies = [
        pltpu.async_copy(vals_v.at[j], out_hbm.at[offs_v.at[j]], ssem)
        for j in range(NSCAT)
    ]
    for cp in scopies:
        cp.wait()


@functools.partial(
    pl.kernel,
    out_type=jax.ShapeDtypeStruct((NB,), jnp.float32),
    mesh=plsc.VectorSubcoreMesh(core_axis_name="c", subcore_axis_name="s"),
    scratch_types=[
        pltpu.VMEM((ZW,), jnp.float32),
        pltpu.VMEM((TH,), jnp.int32),
        pltpu.VMEM((NSCAT, 128), jnp.int32),
        pltpu.VMEM((NSCAT, 128), jnp.float32),
        pltpu.SemaphoreType.DMA,
        pltpu.SemaphoreType.DMA,
    ],
)
def _sc_onehot(idx_hbm, out_hbm, zbuf, idx_v, offs_v, vals_v, zsem, ssem):
    _sc_body(idx_hbm, out_hbm, zbuf, idx_v, offs_v, vals_v, zsem, ssem)


def kernel(quant_onehot, in_snd_slice):
    del quant_onehot
    idx = in_snd_slice.astype(jnp.int32)
    flat = _sc_onehot(idx)
    return flat.reshape(B, N_QUANT, T)


# speedup vs baseline: 2.0353x; 2.0353x over previous
import functools

import jax
import jax.numpy as jnp
from jax import lax
from jax.experimental import pallas as pl
from jax.experimental.pallas import tpu as pltpu
from jax.experimental.pallas import tpu_sc as plsc

N_QUANT = 256
B = 16
T = 8192
QT = N_QUANT * T
T_BLK = 512


def _make_sc_onehot(nrows):
    assert nrows in (2, 4, 8, 16)
    wpr = 32 // nrows
    chunk = QT // wpr
    zsh = min(chunk, 524288)
    nz = chunk // zsh
    zw = zsh // 16
    th = T // wpr
    nscat = th // 128
    nb = nrows * QT

    def body(idx_hbm, out_hbm, zbuf, zshared, idx_v, offs_v, vals_v, zsem, ssem):
        c = lax.axis_index("c")
        s = lax.axis_index("s")
        b = c * (nrows // 2) + s // wpr
        part = s % wpr

        zeros = jnp.zeros((16,), jnp.float32)
        ones = jnp.ones((16,), jnp.float32)

        def fill_zeros(i, carry):
            zbuf[pl.ds(i * 16, 16)] = zeros
            return carry

        lax.fori_loop(0, zw // 16, fill_zeros, 0)
        pltpu.sync_copy(zbuf, zshared.at[pl.ds(s * zw, zw)])

        pltpu.sync_copy(idx_hbm.at[b, pl.ds(part * th, th)], idx_v)
        base = b * QT + part * th
        iota = lax.iota(jnp.int32, 16)
        for j in range(nscat):
            def fill_row(k, carry, j=j):
                v = idx_v[pl.ds(j * 128 + k * 16, 16)]
                offs_v[j, pl.ds(k * 16, 16)] = v * T + (base + j * 128 + k * 16) + iota
                vals_v[j, pl.ds(k * 16, 16)] = ones
                return carry

            lax.fori_loop(0, 8, fill_row, 0)

        plsc.subcore_barrier()

        zbase = b * QT + part * chunk
        zcopies = [
            pltpu.async_copy(zshared, out_hbm.at[pl.ds(zbase + j * zsh, zsh)], zsem)
            for j in range(nz)
        ]
        for cp in zcopies:
            cp.wait()

        plsc.subcore_barrier()

        scopies = [
            pltpu.async_copy(vals_v.at[j], out_hbm.at[offs_v.at[j]], ssem)
            for j in range(nscat)
        ]
        for cp in scopies:
            cp.wait()

    return pl.kernel(
        body,
        out_type=jax.ShapeDtypeStruct((nb,), jnp.float32),
        mesh=plsc.VectorSubcoreMesh(core_axis_name="c", subcore_axis_name="s"),
        scratch_types=[
            pltpu.VMEM((zw,), jnp.float32),
            pltpu.VMEM_SHARED((zsh,), jnp.float32),
            pltpu.VMEM((th,), jnp.int32),
            pltpu.VMEM((nscat, 128), jnp.int32),
            pltpu.VMEM((nscat, 128), jnp.float32),
            pltpu.SemaphoreType.DMA,
            pltpu.SemaphoreType.DMA,
        ],
    )


def _tc_onehot_body(idx_ref, out_ref):
    nb = idx_ref.shape[0]
    idx = idx_ref[...]
    q = jax.lax.broadcasted_iota(jnp.int32, (nb, N_QUANT, T_BLK), 1)
    out_ref[...] = (q == idx[:, None, :]).astype(jnp.float32)


def _tc_onehot(idx):
    nb = idx.shape[0]
    return pl.pallas_call(
        _tc_onehot_body,
        grid=(T // T_BLK,),
        in_specs=[pl.BlockSpec((nb, T_BLK), lambda i: (0, i))],
        out_specs=pl.BlockSpec((nb, N_QUANT, T_BLK), lambda i: (0, 0, i)),
        out_shape=jax.ShapeDtypeStruct((nb, N_QUANT, T), jnp.float32),
    )(idx)


B_SC = 2
_sc_call = _make_sc_onehot(B_SC)


def kernel(quant_onehot, in_snd_slice):
    del quant_onehot
    idx = in_snd_slice.astype(jnp.int32)
    sc_part = _sc_call(idx[:B_SC]).reshape(B_SC, N_QUANT, T)
    tc_part = _tc_onehot(idx[B_SC:])
    return jnp.concatenate([sc_part, tc_part], axis=0)


# speedup vs baseline: 7.9094x; 3.8862x over previous
import jax
import jax.numpy as jnp
from jax.experimental import pallas as pl

N_QUANT = 256
B = 16
T = 8192
T_BLK = 1024


def _onehot_body(idx_ref, out_ref):
    idx = idx_ref[...]
    q = jax.lax.broadcasted_iota(jnp.int32, (B, N_QUANT, T_BLK), 1)
    out_ref[...] = (q == idx[:, None, :]).astype(jnp.float32)


def kernel(quant_onehot, in_snd_slice):
    del quant_onehot
    idx = in_snd_slice.astype(jnp.int32)
    return pl.pallas_call(
        _onehot_body,
        grid=(T // T_BLK,),
        in_specs=[pl.BlockSpec((B, T_BLK), lambda i: (0, i))],
        out_specs=pl.BlockSpec((B, N_QUANT, T_BLK), lambda i: (0, 0, i)),
        out_shape=jax.ShapeDtypeStruct((B, N_QUANT, T), jnp.float32),
    )(idx)


# speedup vs baseline: 8.3874x; 1.0604x over previous
import jax
import jax.numpy as jnp
from jax.experimental import pallas as pl

N_QUANT = 256
B = 16
T = 8192
T_BLK = 256


def _onehot_body(idx_ref, out_ref):
    idx = idx_ref[...]
    q = jax.lax.broadcasted_iota(jnp.int32, (B, N_QUANT, T_BLK), 1)
    out_ref[...] = (q == idx[:, None, :]).astype(jnp.float32)


def kernel(quant_onehot, in_snd_slice):
    del quant_onehot
    idx = in_snd_slice.astype(jnp.int32)
    return pl.pallas_call(
        _onehot_body,
        grid=(T // T_BLK,),
        in_specs=[pl.BlockSpec((B, T_BLK), lambda i: (0, i))],
        out_specs=pl.BlockSpec((B, N_QUANT, T_BLK), lambda i: (0, 0, i)),
        out_shape=jax.ShapeDtypeStruct((B, N_QUANT, T), jnp.float32),
    )(idx)
